# baseline (speedup 1.0000x reference)
import functools

import jax
import jax.numpy as jnp
from jax import lax
from jax.experimental import pallas as pl
from jax.experimental.pallas import tpu as pltpu
from jax.experimental.pallas import tpu_sc as plsc

_BATCH = 4096
_HIST = 200
_DIM = 64
_NCLASS = 100

_NCHUNK = 5
_CHUNK = _HIST // _NCHUNK
_NGRP = _DIM // 16


def _sc_gather_sum(text3, embed_table):
  mesh = plsc.VectorSubcoreMesh(core_axis_name="c", subcore_axis_name="s")
  nw = mesh.num_cores * mesh.num_subcores
  rows_per_w = _BATCH // nw

  @functools.partial(
      pl.kernel,
      out_type=jax.ShapeDtypeStruct((_BATCH, _DIM), jnp.float32),
      mesh=mesh,
      scratch_types=[
          pltpu.VMEM((rows_per_w, _NCHUNK, _CHUNK), jnp.int32),
          pltpu.VMEM((2, _NCHUNK, _CHUNK, _DIM), jnp.float32),
          pltpu.VMEM((rows_per_w, _DIM), jnp.float32),
          pltpu.SemaphoreType.DMA,
          pltpu.SemaphoreType.DMA,
      ],
      compiler_params=pltpu.CompilerParams(use_tc_tiling_on_sc=False),
  )
  def k(text_hbm, table_hbm, out_hbm, idx_v, rows_v, acc_v, sem0, sem1):
    wid = lax.axis_index("s") * mesh.num_cores + lax.axis_index("c")
    base = wid * rows_per_w

    pltpu.sync_copy(text_hbm.at[pl.ds(base, rows_per_w)], idx_v)

    def fire(b, par, sem):
      for j in range(_NCHUNK):
        pltpu.async_copy(table_hbm.at[idx_v.at[b, j]], rows_v.at[par, j], sem)

    def drain(b, par, sem):
      for j in range(_NCHUNK):
        pltpu.make_async_copy(
            table_hbm.at[idx_v.at[b, j]], rows_v.at[par, j], sem).wait()

    def accumulate_and_store(b, par):
      accs = tuple(jnp.zeros((16,), jnp.float32) for _ in range(_NGRP))
      for j in range(_NCHUNK):
        def body(i, accs, j=j):
          accs = list(accs)
          for r in range(8):
            row = i * 8 + r
            for g in range(_NGRP):
              accs[g] = accs[g] + rows_v[par, j, row, pl.ds(g * 16, 16)]
          return tuple(accs)
        accs = lax.fori_loop(0, _CHUNK // 8, body, accs)
      for g in range(_NGRP):
        acc_v[b, pl.ds(g * 16, 16)] = accs[g]

    fire(0, 0, sem0)

    def step(bb, _):
      b0 = 2 * bb
      b1 = 2 * bb + 1
      fire(b1, 1, sem1)
      drain(b0, 0, sem0)

      @pl.when(bb < rows_per_w // 2 - 1)
      def _():
        fire(b0 + 2, 0, sem0)

      accumulate_and_store(b0, 0)
      drain(b1, 1, sem1)
      accumulate_and_store(b1, 1)
      return 0

    lax.fori_loop(0, rows_per_w // 2, step, 0)

    pltpu.sync_copy(acc_v, out_hbm.at[pl.ds(base, rows_per_w)])

  return k(text3, embed_table)


def _tc_linear(sums, fc_weight, fc_bias2):
  def body(x_ref, w_ref, b_ref, o_ref):
    acc = lax.dot_general(
        x_ref[:, :], w_ref[:, :],
        dimension_numbers=(((1,), (1,)), ((), ())),
        preferred_element_type=jnp.float32,
    )
    o_ref[:, :] = acc * (1.0 / _HIST) + b_ref[:, :]

  return pl.pallas_call(
      body,
      out_shape=jax.ShapeDtypeStruct((_BATCH, _NCLASS), jnp.float32),
  )(sums, fc_weight, fc_bias2)


def kernel(text, embed_table, fc_weight, fc_bias):
  text3 = text.astype(jnp.int32).reshape(_BATCH, _NCHUNK, _CHUNK)
  sums = _sc_gather_sum(text3, embed_table)
  return _tc_linear(sums, fc_weight, fc_bias.reshape(1, _NCLASS))
